# baseline (speedup 1.0000x reference)
import functools

import jax
import jax.numpy as jnp
from jax import lax
from jax.experimental import pallas as pl
from jax.experimental.pallas import tpu as pltpu
from jax.experimental.pallas import tpu_sc as plsc

N = 10000
E = 160000
FEATS = 38
D = 512
NCH = 4
CH = 128
ALPHA = 0.2
MAXW = 15.286330223083496

NC = 2
NS = 16
B = 64
NB_T = 160
Q = 32
NQS = NB_T // Q
NR = Q // 4
E_PAD = NS * NB_T * B
N_PAD = 10240
ROWS_PER_TILE = N_PAD // NS
NW = NC * NS
DEG_VECS = E_PAD // NW // 16

_mesh = plsc.VectorSubcoreMesh(
    core_axis_name="c", subcore_axis_name="s", num_cores=NC, num_subcores=NS)
_sc_params = pltpu.CompilerParams(needs_layout_passes=False)


MBX = 2000


def _x0_body(feats_ref, w_ref, b_ref, x0_ref):
    x0_ref[...] = jnp.dot(feats_ref[...], w_ref[...],
                          preferred_element_type=jnp.float32,
                          precision=lax.Precision.HIGHEST) + b_ref[...]


def _x0(feats, lin1_w, lin1_b):
    return pl.pallas_call(
        _x0_body,
        grid=(N // MBX,),
        in_specs=[
            pl.BlockSpec((MBX, FEATS), lambda i: (i, 0)),
            pl.BlockSpec((FEATS, D), lambda i: (0, 0)),
            pl.BlockSpec((1, D), lambda i: (0, 0)),
        ],
        out_specs=pl.BlockSpec((MBX, D), lambda i: (i, 0)),
        out_shape=jax.ShapeDtypeStruct((N, D), jnp.float32),
    )(feats, lin1_w, lin1_b.reshape(1, D))


def _gidx_body(src_ref, gidx_ref):
    s4 = src_ref[...] * NCH
    for c in range(NCH):
        gidx_ref[c] = s4 + c


def _gidx(src2d):
    return pl.pallas_call(
        _gidx_body,
        out_shape=jax.ShapeDtypeStruct((NCH, E_PAD // 128, 128), jnp.int32),
    )(src2d)


def _dinv_body(degt_ref, x0_ref, dinv_ref, hs0_ref):
    deg = jnp.sum(degt_ref[...], axis=1, keepdims=True)
    dinv = jnp.where(deg > 0, lax.rsqrt(jnp.where(deg > 0, deg, 1.0)), 0.0)
    dinv_ref[...] = dinv
    hs0_ref[...] = dinv * x0_ref[...]


def _dinv(degt, x0):
    return pl.pallas_call(
        _dinv_body,
        grid=(N // MBX,),
        in_specs=[
            pl.BlockSpec((MBX, NW), lambda i: (i, 0)),
            pl.BlockSpec((MBX, D), lambda i: (i, 0)),
        ],
        out_specs=[
            pl.BlockSpec((MBX, 1), lambda i: (i, 0)),
            pl.BlockSpec((MBX, D), lambda i: (i, 0)),
        ],
        out_shape=(jax.ShapeDtypeStruct((N, 1), jnp.float32),
                   jax.ShapeDtypeStruct((N, D), jnp.float32)),
    )(degt, x0)


MB = 1000


def _layer_body(s_ref, x0_ref, dinv_ref, w_ref, h_ref, hs_ref):
    dv = dinv_ref[...]
    x0 = x0_ref[...]
    acc = None
    for c in range(NCH):
        mc = (1.0 - ALPHA) * (dv * s_ref[c]) + ALPHA * x0[:, c * CH:(c + 1) * CH]
        p = jnp.dot(mc, w_ref[c * CH:(c + 1) * CH, :],
                    preferred_element_type=jnp.float32,
                    precision=lax.Precision.HIGHEST)
        acc = p if acc is None else acc + p
    h = jnp.maximum(acc, 0.0)
    h_ref[...] = h
    hs_ref[...] = dv * h


def _layer(s4, x0, dinv, w):
    return pl.pallas_call(
        _layer_body,
        grid=(N // MB,),
        in_specs=[
            pl.BlockSpec((NCH, MB, CH), lambda i: (0, i, 0)),
            pl.BlockSpec((MB, D), lambda i: (i, 0)),
            pl.BlockSpec((MB, 1), lambda i: (i, 0)),
            pl.BlockSpec((D, D), lambda i: (0, 0)),
        ],
        out_specs=[
            pl.BlockSpec((MB, D), lambda i: (i, 0)),
            pl.BlockSpec((MB, D), lambda i: (i, 0)),
        ],
        out_shape=(jax.ShapeDtypeStruct((N, D), jnp.float32),
                   jax.ShapeDtypeStruct((N, D), jnp.float32)),
    )(s4, x0, dinv, w)


@functools.partial(
    pl.kernel,
    out_type=jax.ShapeDtypeStruct((NW, N_PAD // 128, 128), jnp.float32),
    mesh=_mesh,
    scratch_types=[
        pltpu.VMEM((DEG_VECS, 16), jnp.int32),
        pltpu.VMEM((DEG_VECS, 16), jnp.float32),
        pltpu.VMEM((N_PAD // 128, 128), jnp.float32),
    ],
    compiler_params=_sc_params,
)
def _deg_kernel(dst_hbm, ew_hbm, degp_hbm, dstb, ewb, degl):
    wid = lax.axis_index("s") * NC + lax.axis_index("c")
    pltpu.sync_copy(dst_hbm.at[wid], dstb)
    pltpu.sync_copy(ew_hbm.at[wid], ewb)

    def zero_body(i, _):
        for p in range(128 // 16):
            degl[i, pl.ds(p * 16, 16)] = jnp.zeros((16,), jnp.float32)
        return 0
    lax.fori_loop(0, N_PAD // 128, zero_body, 0)

    def acc_body(i, _):
        d16 = dstb[i]
        plsc.addupdate_scatter(degl, [d16 >> 7, d16 & 127],
                               ewb[i] * (1.0 / MAXW))
        return 0
    lax.fori_loop(0, DEG_VECS, acc_body, 0)
    pltpu.sync_copy(degl, degp_hbm.at[wid])


@functools.partial(
    pl.kernel,
    out_type=jax.ShapeDtypeStruct((NCH, N_PAD, CH), jnp.float32),
    mesh=_mesh,
    scratch_types=[
        pltpu.VMEM((Q, B), jnp.int32),
        pltpu.VMEM((Q, B), jnp.int32),
        pltpu.VMEM((Q, B), jnp.float32),
        pltpu.VMEM((B, CH), jnp.float32),
        pltpu.VMEM((B, CH), jnp.float32),
        pltpu.VMEM((B, CH), jnp.float32),
        pltpu.VMEM((B, CH), jnp.float32),
        pltpu.VMEM((8, CH), jnp.float32),
        pltpu.VMEM_SHARED((N_PAD, CH), jnp.float32),
        pltpu.SemaphoreType.DMA,
        pltpu.SemaphoreType.DMA,
        pltpu.SemaphoreType.DMA,
        pltpu.SemaphoreType.DMA,
        pltpu.SemaphoreType.DMA,
        pltpu.SemaphoreType.DMA,
        pltpu.SemaphoreType.DMA,
        pltpu.SemaphoreType.DMA,
    ],
    compiler_params=_sc_params,
)
def _seg_kernel(hs_hbm, gidx_hbm, dst_hbm, ew_hbm, out_hbm,
                idxb, dstb, ewb, rows0, rows1, rows2, rows3, zbuf, acc,
                gsem0, gsem1, gsem2, gsem3, ssem0, ssem1, ssem2, ssem3):
    cid = lax.axis_index("c")
    sid = lax.axis_index("s")
    rows = (rows0, rows1, rows2, rows3)
    gsem = (gsem0, gsem1, gsem2, gsem3)
    ssem = (ssem0, ssem1, ssem2, ssem3)

    def zb(i, _):
        for p in range(CH // 16):
            zbuf[i, pl.ds(p * 16, 16)] = jnp.zeros((16,), jnp.float32)
        return 0
    lax.fori_loop(0, 8, zb, 0)

    def scale(buf, b):
        def vec_body(k, _):
            ew16 = ewb[b, pl.ds(k * 16, 16)] * (1.0 / MAXW)
            for lane in range(16):
                sp = jnp.full((16,), ew16[lane], jnp.float32)
                j = k * 16 + lane
                for p in range(CH // 16):
                    buf[j, pl.ds(p * 16, 16)] = buf[j, pl.ds(p * 16, 16)] * sp
            return 0
        lax.fori_loop(0, B // 16, vec_body, 0)

    def wait_gather(buf, sem):
        pltpu.make_async_copy(hs_hbm.at[idxb.at[0]], buf, sem).wait()

    def wait_scatter(buf, sem):
        pltpu.make_async_copy(buf, acc.at[dstb.at[0]], sem).wait()

    row0 = sid * ROWS_PER_TILE
    for ph in range(NCH // NC):
        c = cid * (NCH // NC) + ph
        for z in range(ROWS_PER_TILE // 8):
            pltpu.sync_copy(zbuf, acc.at[pl.ds(row0 + z * 8, 8)])
        plsc.subcore_barrier()

        def qbody(q, _):
            pltpu.sync_copy(gidx_hbm.at[c, sid, q], idxb)
            pltpu.sync_copy(dst_hbm.at[sid, q], dstb)
            pltpu.sync_copy(ew_hbm.at[sid, q], ewb)
            pltpu.async_copy(hs_hbm.at[idxb.at[0]], rows[0], gsem[0])
            pltpu.async_copy(hs_hbm.at[idxb.at[1]], rows[1], gsem[1])

            def rbody(r, _):
                for s in range(4):
                    bb = r * 4 + s
                    wait_gather(rows[s], gsem[s])
                    scale(rows[s], bb)
                    pltpu.async_copy(rows[s], acc.at[dstb.at[bb]],
                                     ssem[s], add=True)
                    s2 = (s + 2) % 4

                    @pl.when(bb >= 2)
                    def _ws():
                        wait_scatter(rows[s2], ssem[s2])

                    @pl.when(bb + 2 < Q)
                    def _pf():
                        pltpu.async_copy(hs_hbm.at[idxb.at[bb + 2]],
                                         rows[s2], gsem[s2])
                return 0
            lax.fori_loop(0, NR, rbody, 0)
            wait_scatter(rows[2], ssem[2])
            wait_scatter(rows[3], ssem[3])
            return 0
        lax.fori_loop(0, NQS, qbody, 0)
        plsc.subcore_barrier()
        pltpu.sync_copy(acc.at[pl.ds(row0, ROWS_PER_TILE)],
                        out_hbm.at[c, pl.ds(row0, ROWS_PER_TILE)])


def kernel(x, edge_index, edge_attr, lin1_w, lin1_b, w1, w2, w3, lin2_w, lin2_b):
    feats = x[:, :FEATS]
    src = edge_index[0]
    dst = edge_index[1]
    ea = edge_attr[:, 3]
    pad = E_PAD - E
    src_p = jnp.concatenate([src, jnp.zeros((pad,), jnp.int32)])
    dst_p = jnp.concatenate([dst, jnp.zeros((pad,), jnp.int32)])
    ea_p = jnp.concatenate([ea, jnp.zeros((pad,), jnp.float32)])

    x0 = _x0(feats, lin1_w, lin1_b)
    degp = _deg_kernel(dst_p.reshape(NW, DEG_VECS, 16),
                       ea_p.reshape(NW, DEG_VECS, 16))
    dinv, hs = _dinv(degp.reshape(NW, N_PAD)[:, :N].T, x0)

    gidx = _gidx(src_p.reshape(E_PAD // 128, 128))
    gidx_r = gidx.reshape(NCH, NS, NQS, Q, B)
    dst_r = dst_p.reshape(NS, NQS, Q, B)
    ew_r = ea_p.reshape(NS, NQS, Q, B)
    h = None
    for w in (w1, w2, w3):
        s4 = _seg_kernel(hs.reshape(N * NCH, CH), gidx_r, dst_r, ew_r)
        h, hs = _layer(s4, x0, dinv, w)
    return h


# speedup vs baseline: 1.0300x; 1.0300x over previous
import functools

import jax
import jax.numpy as jnp
from jax import lax
from jax.experimental import pallas as pl
from jax.experimental.pallas import tpu as pltpu
from jax.experimental.pallas import tpu_sc as plsc

N = 10000
E = 160000
FEATS = 38
D = 512
NCH = 4
CH = 128
ALPHA = 0.2
MAXW = 15.286330223083496

NC = 2
NS = 16
B = 64
NB_T = 160
Q = 32
NQS = NB_T // Q
NR = Q // 4
E_PAD = NS * NB_T * B
N_PAD = 10240
ROWS_PER_TILE = N_PAD // NS
NW = NC * NS
DEG_VECS = E_PAD // NW // 16

_mesh = plsc.VectorSubcoreMesh(
    core_axis_name="c", subcore_axis_name="s", num_cores=NC, num_subcores=NS)
_sc_params = pltpu.CompilerParams(needs_layout_passes=False)


MBX = 2000


def _x0_body(feats_ref, w_ref, b_ref, x0_ref):
    x0_ref[...] = jnp.dot(feats_ref[...], w_ref[...],
                          preferred_element_type=jnp.float32,
                          precision=lax.Precision.HIGHEST) + b_ref[...]


def _x0(feats, lin1_w, lin1_b):
    return pl.pallas_call(
        _x0_body,
        grid=(N // MBX,),
        in_specs=[
            pl.BlockSpec((MBX, FEATS), lambda i: (i, 0)),
            pl.BlockSpec((FEATS, D), lambda i: (0, 0)),
            pl.BlockSpec((1, D), lambda i: (0, 0)),
        ],
        out_specs=pl.BlockSpec((MBX, D), lambda i: (i, 0)),
        out_shape=jax.ShapeDtypeStruct((N, D), jnp.float32),
    )(feats, lin1_w, lin1_b.reshape(1, D))


def _gidx_body(src_ref, gidx_ref):
    s4 = src_ref[...] * NCH
    for c in range(NCH):
        gidx_ref[c] = s4 + c


def _gidx(src2d):
    return pl.pallas_call(
        _gidx_body,
        out_shape=jax.ShapeDtypeStruct((NCH, E_PAD // 128, 128), jnp.int32),
    )(src2d)


def _dinv_body(degt_ref, x0_ref, dinv_ref, hs0_ref):
    deg = jnp.sum(degt_ref[...], axis=1, keepdims=True)
    dinv = jnp.where(deg > 0, lax.rsqrt(jnp.where(deg > 0, deg, 1.0)), 0.0)
    dinv_ref[...] = dinv
    hs0_ref[...] = dinv * x0_ref[...]


def _dinv(degt, x0):
    return pl.pallas_call(
        _dinv_body,
        grid=(N // MBX,),
        in_specs=[
            pl.BlockSpec((MBX, NW), lambda i: (i, 0)),
            pl.BlockSpec((MBX, D), lambda i: (i, 0)),
        ],
        out_specs=[
            pl.BlockSpec((MBX, 1), lambda i: (i, 0)),
            pl.BlockSpec((MBX, D), lambda i: (i, 0)),
        ],
        out_shape=(jax.ShapeDtypeStruct((N, 1), jnp.float32),
                   jax.ShapeDtypeStruct((N, D), jnp.float32)),
    )(degt, x0)


MB = 1000


def _layer_body(s_ref, x0_ref, dinv_ref, w_ref, h_ref, hs_ref):
    dv = dinv_ref[...]
    x0 = x0_ref[...]
    acc = None
    for c in range(NCH):
        mc = (1.0 - ALPHA) * (dv * s_ref[c]) + ALPHA * x0[:, c * CH:(c + 1) * CH]
        p = jnp.dot(mc, w_ref[c * CH:(c + 1) * CH, :],
                    preferred_element_type=jnp.float32,
                    precision=lax.Precision.HIGHEST)
        acc = p if acc is None else acc + p
    h = jnp.maximum(acc, 0.0)
    h_ref[...] = h
    hs_ref[...] = dv * h


def _layer(s4, x0, dinv, w):
    return pl.pallas_call(
        _layer_body,
        grid=(N // MB,),
        in_specs=[
            pl.BlockSpec((NCH, MB, CH), lambda i: (0, i, 0)),
            pl.BlockSpec((MB, D), lambda i: (i, 0)),
            pl.BlockSpec((MB, 1), lambda i: (i, 0)),
            pl.BlockSpec((D, D), lambda i: (0, 0)),
        ],
        out_specs=[
            pl.BlockSpec((MB, D), lambda i: (i, 0)),
            pl.BlockSpec((MB, D), lambda i: (i, 0)),
        ],
        out_shape=(jax.ShapeDtypeStruct((N, D), jnp.float32),
                   jax.ShapeDtypeStruct((N, D), jnp.float32)),
    )(s4, x0, dinv, w)


@functools.partial(
    pl.kernel,
    out_type=jax.ShapeDtypeStruct((NW, N_PAD // 128, 128), jnp.float32),
    mesh=_mesh,
    scratch_types=[
        pltpu.VMEM((DEG_VECS, 16), jnp.int32),
        pltpu.VMEM((DEG_VECS, 16), jnp.float32),
        pltpu.VMEM((N_PAD // 128, 128), jnp.float32),
    ],
    compiler_params=_sc_params,
)
def _deg_kernel(dst_hbm, ew_hbm, degp_hbm, dstb, ewb, degl):
    wid = lax.axis_index("s") * NC + lax.axis_index("c")
    pltpu.sync_copy(dst_hbm.at[wid], dstb)
    pltpu.sync_copy(ew_hbm.at[wid], ewb)

    def zero_body(i, _):
        for p in range(128 // 16):
            degl[i, pl.ds(p * 16, 16)] = jnp.zeros((16,), jnp.float32)
        return 0
    lax.fori_loop(0, N_PAD // 128, zero_body, 0)

    def acc_body(i, _):
        d16 = dstb[i]
        plsc.addupdate_scatter(degl, [d16 >> 7, d16 & 127],
                               ewb[i] * (1.0 / MAXW))
        return 0
    lax.fori_loop(0, DEG_VECS, acc_body, 0)
    pltpu.sync_copy(degl, degp_hbm.at[wid])


@functools.partial(
    pl.kernel,
    out_type=jax.ShapeDtypeStruct((NCH, N_PAD, CH), jnp.float32),
    mesh=_mesh,
    scratch_types=[
        pltpu.VMEM((Q, B), jnp.int32),
        pltpu.VMEM((Q, B), jnp.int32),
        pltpu.VMEM((Q, B), jnp.float32),
        pltpu.VMEM((B, CH), jnp.float32),
        pltpu.VMEM((B, CH), jnp.float32),
        pltpu.VMEM((B, CH), jnp.float32),
        pltpu.VMEM((B, CH), jnp.float32),
        pltpu.VMEM((8, CH), jnp.float32),
        pltpu.VMEM_SHARED((N_PAD, CH), jnp.float32),
        pltpu.SemaphoreType.DMA,
        pltpu.SemaphoreType.DMA,
        pltpu.SemaphoreType.DMA,
        pltpu.SemaphoreType.DMA,
        pltpu.SemaphoreType.DMA,
        pltpu.SemaphoreType.DMA,
        pltpu.SemaphoreType.DMA,
        pltpu.SemaphoreType.DMA,
    ],
    compiler_params=_sc_params,
)
def _seg_kernel(hs_hbm, gidx_hbm, dst_hbm, ew_hbm, out_hbm,
                idxb, dstb, ewb, rows0, rows1, rows2, rows3, zbuf, acc,
                gsem0, gsem1, gsem2, gsem3, ssem0, ssem1, ssem2, ssem3):
    cid = lax.axis_index("c")
    sid = lax.axis_index("s")
    rows = (rows0, rows1, rows2, rows3)
    gsem = (gsem0, gsem1, gsem2, gsem3)
    ssem = (ssem0, ssem1, ssem2, ssem3)

    def zb(i, _):
        for p in range(CH // 16):
            zbuf[i, pl.ds(p * 16, 16)] = jnp.zeros((16,), jnp.float32)
        return 0
    lax.fori_loop(0, 8, zb, 0)

    def scale(buf, b):
        def vec_body(k, _):
            ew16 = ewb[b, pl.ds(k * 16, 16)] * (1.0 / MAXW)
            for lane in range(16):
                sp = jnp.full((16,), ew16[lane], jnp.float32)
                j = k * 16 + lane
                for p in range(CH // 16):
                    buf[j, pl.ds(p * 16, 16)] = buf[j, pl.ds(p * 16, 16)] * sp
            return 0
        lax.fori_loop(0, B // 16, vec_body, 0)

    def wait_gather(buf, sem):
        pltpu.make_async_copy(hs_hbm.at[idxb.at[0]], buf, sem).wait()

    def wait_scatter(buf, sem):
        pltpu.make_async_copy(buf, acc.at[dstb.at[0]], sem).wait()

    row0 = sid * ROWS_PER_TILE
    for ph in range(NCH // NC):
        c = cid * (NCH // NC) + ph
        for z in range(ROWS_PER_TILE // 8):
            pltpu.sync_copy(zbuf, acc.at[pl.ds(row0 + z * 8, 8)])
        plsc.subcore_barrier()

        def qbody(q, _):
            pltpu.sync_copy(gidx_hbm.at[c, sid, q], idxb)
            pltpu.sync_copy(dst_hbm.at[sid, q], dstb)
            pltpu.sync_copy(ew_hbm.at[sid, q], ewb)
            pltpu.async_copy(hs_hbm.at[idxb.at[0]], rows[0], gsem[0])
            pltpu.async_copy(hs_hbm.at[idxb.at[1]], rows[1], gsem[1])
            pltpu.async_copy(hs_hbm.at[idxb.at[2]], rows[2], gsem[2])

            def rbody(r, _):
                for s in range(4):
                    bb = r * 4 + s
                    wait_gather(rows[s], gsem[s])
                    scale(rows[s], bb)
                    pltpu.async_copy(rows[s], acc.at[dstb.at[bb]],
                                     ssem[s], add=True)
                    s3 = (s + 3) % 4

                    @pl.when(bb >= 1)
                    def _ws():
                        wait_scatter(rows[s3], ssem[s3])

                    @pl.when(bb + 3 < Q)
                    def _pf():
                        pltpu.async_copy(hs_hbm.at[idxb.at[bb + 3]],
                                         rows[s3], gsem[s3])
                return 0
            lax.fori_loop(0, NR, rbody, 0)
            wait_scatter(rows[3], ssem[3])
            return 0
        lax.fori_loop(0, NQS, qbody, 0)
        plsc.subcore_barrier()
        pltpu.sync_copy(acc.at[pl.ds(row0, ROWS_PER_TILE)],
                        out_hbm.at[c, pl.ds(row0, ROWS_PER_TILE)])


def kernel(x, edge_index, edge_attr, lin1_w, lin1_b, w1, w2, w3, lin2_w, lin2_b):
    feats = x[:, :FEATS]
    src = edge_index[0]
    dst = edge_index[1]
    ea = edge_attr[:, 3]
    pad = E_PAD - E
    src_p = jnp.concatenate([src, jnp.zeros((pad,), jnp.int32)])
    dst_p = jnp.concatenate([dst, jnp.zeros((pad,), jnp.int32)])
    ea_p = jnp.concatenate([ea, jnp.zeros((pad,), jnp.float32)])

    x0 = _x0(feats, lin1_w, lin1_b)
    degp = _deg_kernel(dst_p.reshape(NW, DEG_VECS, 16),
                       ea_p.reshape(NW, DEG_VECS, 16))
    dinv, hs = _dinv(degp.reshape(NW, N_PAD)[:, :N].T, x0)

    gidx = _gidx(src_p.reshape(E_PAD // 128, 128))
    gidx_r = gidx.reshape(NCH, NS, NQS, Q, B)
    dst_r = dst_p.reshape(NS, NQS, Q, B)
    ew_r = ea_p.reshape(NS, NQS, Q, B)
    h = None
    for w in (w1, w2, w3):
        s4 = _seg_kernel(hs.reshape(N * NCH, CH), gidx_r, dst_r, ew_r)
        h, hs = _layer(s4, x0, dinv, w)
    return h
